# Initial kernel scaffold; baseline (speedup 1.0000x reference)
import jax
import jax.numpy as jnp
from jax.experimental import pallas as pl


def kernel(input, kernel):
    raise NotImplementedError("write your pallas kernel here")


# speedup vs baseline: 12.4451x; 12.4451x over previous
import jax
import jax.numpy as jnp
from jax.experimental import pallas as pl
from jax.experimental.pallas import tpu as pltpu

_ROWS = 512
_COLS = 512
_KS = 5
_WI = _KS // 2


def _splat_body(kw_ref, x_ref, o_ref):
    acc = None
    for a in range(_KS):
        u = 2 * _WI - a
        for b in range(_KS):
            v = 2 * _WI - b
            term = kw_ref[a, b] * x_ref[u:u + _ROWS, v:v + _COLS]
            acc = term if acc is None else acc + term
    o_ref[...] = acc


def kernel(input, kernel):
    padded = jnp.zeros((_ROWS + 2 * _WI, _COLS + 2 * _WI), dtype=input.dtype)
    padded = jax.lax.dynamic_update_slice(padded, input, (_WI, _WI))
    return pl.pallas_call(
        _splat_body,
        out_shape=jax.ShapeDtypeStruct((_ROWS, _COLS), input.dtype),
        in_specs=[
            pl.BlockSpec(memory_space=pltpu.SMEM),
            pl.BlockSpec((_ROWS + 2 * _WI, _COLS + 2 * _WI), lambda: (0, 0)),
        ],
        out_specs=pl.BlockSpec((_ROWS, _COLS), lambda: (0, 0)),
    )(kernel, padded)


# speedup vs baseline: 27.5460x; 2.2134x over previous
import jax
import jax.numpy as jnp
from jax.experimental import pallas as pl
from jax.experimental.pallas import tpu as pltpu

_ROWS = 512
_COLS = 512
_KS = 5
_WI = _KS // 2


def _splat_body(kw_ref, x_ref, o_ref, s_ref, r_ref):
    x = x_ref[...]
    for v in range(_KS):
        d = v - _WI
        if d < 0:
            sv = jnp.concatenate(
                [jnp.zeros((_ROWS, -d), jnp.float32), x[:, :_COLS + d]], axis=1)
        elif d > 0:
            sv = jnp.concatenate(
                [x[:, d:], jnp.zeros((_ROWS, d), jnp.float32)], axis=1)
        else:
            sv = x
        s_ref[v, :, :] = sv
    for a in range(_KS):
        ra = None
        for b in range(_KS):
            term = kw_ref[a, b] * s_ref[2 * _WI - b, :, :]
            ra = term if ra is None else ra + term
        r_ref[a, :_WI, :] = jnp.zeros((_WI, _COLS), jnp.float32)
        r_ref[a, _WI:_WI + _ROWS, :] = ra
        r_ref[a, _WI + _ROWS:, :] = jnp.zeros((_WI, _COLS), jnp.float32)
    acc = None
    for a in range(_KS):
        u = 2 * _WI - a
        term = r_ref[a, u:u + _ROWS, :]
        acc = term if acc is None else acc + term
    o_ref[...] = acc


def kernel(input, kernel):
    pad_rows = _ROWS + 2 * _WI
    return pl.pallas_call(
        _splat_body,
        out_shape=jax.ShapeDtypeStruct((_ROWS, _COLS), input.dtype),
        in_specs=[
            pl.BlockSpec(memory_space=pltpu.SMEM),
            pl.BlockSpec((_ROWS, _COLS), lambda: (0, 0)),
        ],
        out_specs=pl.BlockSpec((_ROWS, _COLS), lambda: (0, 0)),
        scratch_shapes=[
            pltpu.VMEM((_KS, _ROWS, _COLS), jnp.float32),
            pltpu.VMEM((_KS, pad_rows, _COLS), jnp.float32),
        ],
    )(kernel, input)


# speedup vs baseline: 29.0545x; 1.0548x over previous
import jax
import jax.numpy as jnp
from jax.experimental import pallas as pl
from jax.experimental.pallas import tpu as pltpu

_ROWS = 512
_COLS = 512
_KS = 5
_WI = _KS // 2


def _splat_body(kw_ref, x_ref, o_ref, r_ref):
    x = x_ref[...]
    ras = [None] * _KS
    for v in range(_KS):
        d = v - _WI
        if d < 0:
            sv = jnp.concatenate(
                [jnp.zeros((_ROWS, -d), jnp.float32), x[:, :_COLS + d]], axis=1)
        elif d > 0:
            sv = jnp.concatenate(
                [x[:, d:], jnp.zeros((_ROWS, d), jnp.float32)], axis=1)
        else:
            sv = x
        b = 2 * _WI - v
        for a in range(_KS):
            term = kw_ref[a, b] * sv
            ras[a] = term if ras[a] is None else ras[a] + term
    for a in range(_KS):
        r_ref[a, :_WI, :] = jnp.zeros((_WI, _COLS), jnp.float32)
        r_ref[a, _WI:_WI + _ROWS, :] = ras[a]
        r_ref[a, _WI + _ROWS:, :] = jnp.zeros((_WI, _COLS), jnp.float32)
    acc = None
    for a in range(_KS):
        u = 2 * _WI - a
        term = r_ref[a, u:u + _ROWS, :]
        acc = term if acc is None else acc + term
    o_ref[...] = acc


def kernel(input, kernel):
    pad_rows = _ROWS + 2 * _WI
    return pl.pallas_call(
        _splat_body,
        out_shape=jax.ShapeDtypeStruct((_ROWS, _COLS), input.dtype),
        in_specs=[
            pl.BlockSpec(memory_space=pltpu.SMEM),
            pl.BlockSpec((_ROWS, _COLS), lambda: (0, 0)),
        ],
        out_specs=pl.BlockSpec((_ROWS, _COLS), lambda: (0, 0)),
        scratch_shapes=[
            pltpu.VMEM((_KS, pad_rows, _COLS), jnp.float32),
        ],
    )(kernel, input)
